# baseline (speedup 1.0000x reference)
import functools
import jax
import jax.numpy as jnp
from jax import lax
from jax.experimental import pallas as pl
from jax.experimental.pallas import tpu as pltpu
from jax.experimental.pallas import tpu_sc as plsc

BATCH = 16384
EMB = 64
NC, NS = 2, 16
NW = NC * NS
B_PER_W = BATCH // NW
CH = 128
NCH = B_PER_W // CH
IDROWS = BATCH // CH

_sc_mesh = plsc.VectorSubcoreMesh(core_axis_name="c", subcore_axis_name="s")


@functools.partial(
    pl.kernel,
    out_type=[jax.ShapeDtypeStruct((BATCH, 2 * EMB), jnp.float32)] * 2,
    mesh=_sc_mesh,
    compiler_params=pltpu.CompilerParams(use_tc_tiling_on_sc=True),
    scratch_types=[
        pltpu.VMEM((NCH, CH), jnp.int32),
        pltpu.VMEM((NCH, CH), jnp.int32),
        pltpu.VMEM((CH, 2 * EMB), jnp.float32),
        pltpu.VMEM((CH, 2 * EMB), jnp.float32),
        pltpu.VMEM((CH, 2 * EMB), jnp.float32),
        pltpu.VMEM((CH, 2 * EMB), jnp.float32),
        pltpu.SemaphoreType.DMA,
        pltpu.SemaphoreType.DMA,
    ],
)
def _sc_gather(uid_hbm, iid_hbm, utab_hbm, itab_hbm,
               out_u, out_i, uidx, iidx, bu0, bu1, bi0, bi1, semg, semw):
    wid = lax.axis_index("s") * NC + lax.axis_index("c")
    base = wid * B_PER_W
    pltpu.sync_copy(uid_hbm.at[pl.ds(wid * NCH, NCH)], uidx)
    pltpu.sync_copy(iid_hbm.at[pl.ds(wid * NCH, NCH)], iidx)

    ubufs, ibufs = (bu0, bu1), (bi0, bi1)
    gath = []
    for k in range(NCH):
        bu, bi = ubufs[k % 2], ibufs[k % 2]
        gath.append((
            pltpu.async_copy(utab_hbm.at[uidx.at[k]], bu, semg),
            pltpu.async_copy(itab_hbm.at[iidx.at[k]], bi, semg),
        ))
        if k >= 1:
            pbu, pbi = ubufs[(k - 1) % 2], ibufs[(k - 1) % 2]
            for cp in gath[k - 1]:
                cp.wait()
            orows = pl.ds(base + (k - 1) * CH, CH)
            pltpu.async_copy(pbu, out_u.at[orows], semw)
            pltpu.async_copy(pbi, out_i.at[orows], semw)
    for cp in gath[NCH - 1]:
        cp.wait()
    orows = pl.ds(base + (NCH - 1) * CH, CH)
    wu = pltpu.async_copy(ubufs[(NCH - 1) % 2], out_u.at[orows], semw)
    wi = pltpu.async_copy(ibufs[(NCH - 1) % 2], out_i.at[orows], semw)
    for k in range(NCH - 1):
        orows = pl.ds(base + k * CH, CH)
        pltpu.make_async_copy(ubufs[k % 2], out_u.at[orows], semw).wait()
        pltpu.make_async_copy(ibufs[k % 2], out_i.at[orows], semw).wait()
    wu.wait()
    wi.wait()


N_ROWS = 100000
RC = 2000


def _concat_body(gu, mu, gi, mi, outu, outi):
    outu[:, :EMB] = gu[:]
    outu[:, EMB:] = mu[:]
    outi[:, :EMB] = gi[:]
    outi[:, EMB:] = mi[:]


_tc_concat = pl.pallas_call(
    _concat_body,
    grid=(N_ROWS // RC,),
    in_specs=[pl.BlockSpec((RC, EMB), lambda i: (i, 0))] * 4,
    out_specs=[pl.BlockSpec((RC, 2 * EMB), lambda i: (i, 0))] * 2,
    out_shape=[jax.ShapeDtypeStruct((N_ROWS, 2 * EMB), jnp.float32)] * 2,
)


BB = 4096


def _tc_mlp_body(u, it, w1a, w1b, b1, w2, b2, w3, b3, wog, woh, bo, out):
    f32 = jnp.float32
    uu = u[:]
    ii = it[:]
    g = uu[:, :EMB] * ii[:, :EMB]
    acc = jnp.dot(g, wog[:], preferred_element_type=f32)
    h = jnp.dot(uu[:, EMB:], w1a[:], preferred_element_type=f32)
    h = h + jnp.dot(ii[:, EMB:], w1b[:], preferred_element_type=f32)
    h = jnp.maximum(h + b1[:], 0.0)
    h = jnp.maximum(jnp.dot(h, w2[:], preferred_element_type=f32) + b2[:], 0.0)
    h = jnp.maximum(jnp.dot(h, w3[:], preferred_element_type=f32) + b3[:], 0.0)
    out[:] = acc + jnp.dot(h, woh[:], preferred_element_type=f32) + bo[0, 0]


def _row_spec():
    return pl.BlockSpec((BB, 2 * EMB), lambda i: (i, 0))


def _full_spec(shape):
    return pl.BlockSpec(shape, lambda i: tuple(0 for _ in shape))


_tc_mlp = pl.pallas_call(
    _tc_mlp_body,
    grid=(BATCH // BB,),
    in_specs=[
        _row_spec(), _row_spec(),
        _full_spec((EMB, 128)), _full_spec((EMB, 128)), _full_spec((1, 128)),
        _full_spec((128, 64)), _full_spec((1, 64)),
        _full_spec((64, 32)), _full_spec((1, 32)),
        _full_spec((EMB, 1)), _full_spec((32, 1)), _full_spec((1, 1)),
    ],
    out_specs=pl.BlockSpec((BB, 1), lambda i: (i, 0)),
    out_shape=jax.ShapeDtypeStruct((BATCH, 1), jnp.float32),
)


@jax.jit
def kernel(user_ids, item_ids, gmf_user_emb, gmf_item_emb, mlp_user_emb,
           mlp_item_emb, W1, b1, W2, b2, W3, b3, Wo, bo):
    uid2d = user_ids.astype(jnp.int32).reshape(IDROWS, CH)
    iid2d = item_ids.astype(jnp.int32).reshape(IDROWS, CH)
    utab, itab = _tc_concat(gmf_user_emb, mlp_user_emb,
                            gmf_item_emb, mlp_item_emb)
    rows_u, rows_i = _sc_gather(uid2d, iid2d, utab, itab)
    pred = _tc_mlp(rows_u, rows_i,
                   W1[:EMB], W1[EMB:], b1.reshape(1, -1),
                   W2, b2.reshape(1, -1), W3, b3.reshape(1, -1),
                   Wo[:EMB], Wo[EMB:], bo.reshape(1, 1))
    return pred.reshape(BATCH)


# speedup vs baseline: 1.2206x; 1.2206x over previous
import functools
import jax
import jax.numpy as jnp
from jax import lax
from jax.experimental import pallas as pl
from jax.experimental.pallas import tpu as pltpu
from jax.experimental.pallas import tpu_sc as plsc

BATCH = 16384
EMB = 64
NC, NS = 2, 16
NW = NC * NS
B_PER_W = BATCH // NW
CH = 128
NCH = B_PER_W // CH
IDROWS = BATCH // CH

_sc_mesh = plsc.VectorSubcoreMesh(core_axis_name="c", subcore_axis_name="s")


@functools.partial(
    pl.kernel,
    out_type=[jax.ShapeDtypeStruct((BATCH, 2 * EMB), jnp.float32)] * 2,
    mesh=_sc_mesh,
    compiler_params=pltpu.CompilerParams(use_tc_tiling_on_sc=True),
    scratch_types=[
        pltpu.VMEM((NCH, CH), jnp.int32),
        pltpu.VMEM((NCH, CH), jnp.int32),
        pltpu.VMEM((CH, 2 * EMB), jnp.float32),
        pltpu.VMEM((CH, 2 * EMB), jnp.float32),
        pltpu.VMEM((CH, 2 * EMB), jnp.float32),
        pltpu.VMEM((CH, 2 * EMB), jnp.float32),
        pltpu.SemaphoreType.DMA,
        pltpu.SemaphoreType.DMA,
    ],
)
def _sc_gather(uid_hbm, iid_hbm, utab_hbm, itab_hbm,
               out_u, out_i, uidx, iidx, bu0, bu1, bi0, bi1, semg, semw):
    wid = lax.axis_index("s") * NC + lax.axis_index("c")
    base = wid * B_PER_W
    pltpu.sync_copy(uid_hbm.at[pl.ds(wid * NCH, NCH)], uidx)
    pltpu.sync_copy(iid_hbm.at[pl.ds(wid * NCH, NCH)], iidx)

    ubufs, ibufs = (bu0, bu1), (bi0, bi1)
    gath = []
    for k in range(NCH):
        bu, bi = ubufs[k % 2], ibufs[k % 2]
        gath.append((
            pltpu.async_copy(utab_hbm.at[uidx.at[k]], bu, semg),
            pltpu.async_copy(itab_hbm.at[iidx.at[k]], bi, semg),
        ))
        if k >= 1:
            pbu, pbi = ubufs[(k - 1) % 2], ibufs[(k - 1) % 2]
            for cp in gath[k - 1]:
                cp.wait()
            orows = pl.ds(base + (k - 1) * CH, CH)
            pltpu.async_copy(pbu, out_u.at[orows], semw)
            pltpu.async_copy(pbi, out_i.at[orows], semw)
    for cp in gath[NCH - 1]:
        cp.wait()
    orows = pl.ds(base + (NCH - 1) * CH, CH)
    wu = pltpu.async_copy(ubufs[(NCH - 1) % 2], out_u.at[orows], semw)
    wi = pltpu.async_copy(ibufs[(NCH - 1) % 2], out_i.at[orows], semw)
    for k in range(NCH - 1):
        orows = pl.ds(base + k * CH, CH)
        pltpu.make_async_copy(ubufs[k % 2], out_u.at[orows], semw).wait()
        pltpu.make_async_copy(ibufs[k % 2], out_i.at[orows], semw).wait()
    wu.wait()
    wi.wait()


N_ROWS = 100000
RC = 1000


def _concat_body(gu, mu, gi, mi, outu, outi):
    outu[:, :EMB] = gu[:].T
    outu[:, EMB:] = mu[:].T
    outi[:, :EMB] = gi[:].T
    outi[:, EMB:] = mi[:].T


_tc_concat = pl.pallas_call(
    _concat_body,
    grid=(N_ROWS // RC,),
    in_specs=[pl.BlockSpec((EMB, RC), lambda i: (0, i))] * 4,
    out_specs=[pl.BlockSpec((RC, 2 * EMB), lambda i: (i, 0))] * 2,
    out_shape=[jax.ShapeDtypeStruct((N_ROWS, 2 * EMB), jnp.float32)] * 2,
)


BB = 4096


def _tc_mlp_body(u, it, w1a, w1b, b1, w2, b2, w3, b3, wog, woh, bo, out):
    f32 = jnp.float32
    uu = u[:]
    ii = it[:]
    g = uu[:, :EMB] * ii[:, :EMB]
    acc = jnp.dot(g, wog[:], preferred_element_type=f32)
    h = jnp.dot(uu[:, EMB:], w1a[:], preferred_element_type=f32)
    h = h + jnp.dot(ii[:, EMB:], w1b[:], preferred_element_type=f32)
    h = jnp.maximum(h + b1[:], 0.0)
    h = jnp.maximum(jnp.dot(h, w2[:], preferred_element_type=f32) + b2[:], 0.0)
    h = jnp.maximum(jnp.dot(h, w3[:], preferred_element_type=f32) + b3[:], 0.0)
    out[:] = acc + jnp.dot(h, woh[:], preferred_element_type=f32) + bo[0, 0]


def _row_spec():
    return pl.BlockSpec((BB, 2 * EMB), lambda i: (i, 0))


def _full_spec(shape):
    return pl.BlockSpec(shape, lambda i: tuple(0 for _ in shape))


_tc_mlp = pl.pallas_call(
    _tc_mlp_body,
    grid=(BATCH // BB,),
    in_specs=[
        _row_spec(), _row_spec(),
        _full_spec((EMB, 128)), _full_spec((EMB, 128)), _full_spec((1, 128)),
        _full_spec((128, 64)), _full_spec((1, 64)),
        _full_spec((64, 32)), _full_spec((1, 32)),
        _full_spec((EMB, 1)), _full_spec((32, 1)), _full_spec((1, 1)),
    ],
    out_specs=pl.BlockSpec((BB, 1), lambda i: (i, 0)),
    out_shape=jax.ShapeDtypeStruct((BATCH, 1), jnp.float32),
)


@jax.jit
def kernel(user_ids, item_ids, gmf_user_emb, gmf_item_emb, mlp_user_emb,
           mlp_item_emb, W1, b1, W2, b2, W3, b3, Wo, bo):
    uid2d = user_ids.astype(jnp.int32).reshape(IDROWS, CH)
    iid2d = item_ids.astype(jnp.int32).reshape(IDROWS, CH)
    utabT = jnp.concatenate([gmf_user_emb.T, mlp_user_emb.T], axis=0)
    itabT = jnp.concatenate([gmf_item_emb.T, mlp_item_emb.T], axis=0)
    utabT, itabT = jax.lax.optimization_barrier((utabT, itabT))
    utab = utabT.T
    itab = itabT.T
    rows_u, rows_i = _sc_gather(uid2d, iid2d, utab, itab)
    pred = _tc_mlp(rows_u, rows_i,
                   W1[:EMB], W1[EMB:], b1.reshape(1, -1),
                   W2, b2.reshape(1, -1), W3, b3.reshape(1, -1),
                   Wo[:EMB], Wo[EMB:], bo.reshape(1, 1))
    return pred.reshape(BATCH)


# speedup vs baseline: 1.2784x; 1.0474x over previous
import functools
import jax
import jax.numpy as jnp
from jax import lax
from jax.experimental import pallas as pl
from jax.experimental.pallas import tpu as pltpu
from jax.experimental.pallas import tpu_sc as plsc

BATCH = 16384
EMB = 64
NC, NS = 2, 16
NW = NC * NS
B_PER_W = BATCH // NW
CH = 128
NCH = B_PER_W // CH
IDROWS = BATCH // CH

_sc_mesh = plsc.VectorSubcoreMesh(core_axis_name="c", subcore_axis_name="s")


@functools.partial(
    pl.kernel,
    out_type=jax.ShapeDtypeStruct((BATCH, 2 * EMB), jnp.float32),
    mesh=_sc_mesh,
    compiler_params=pltpu.CompilerParams(use_tc_tiling_on_sc=True),
    scratch_types=[
        pltpu.VMEM((NCH, CH), jnp.int32),
        pltpu.VMEM((CH, 2 * EMB), jnp.float32),
        pltpu.VMEM((CH, 2 * EMB), jnp.float32),
        pltpu.SemaphoreType.DMA,
        pltpu.SemaphoreType.DMA,
    ],
)
def _sc_gather(id_hbm, tab_hbm, out, idx, b0, b1, semg, semw):
    wid = lax.axis_index("s") * NC + lax.axis_index("c")
    base = wid * B_PER_W
    pltpu.sync_copy(id_hbm.at[pl.ds(wid * NCH, NCH)], idx)

    bufs = (b0, b1)
    gath = []
    for k in range(NCH):
        gath.append(pltpu.async_copy(tab_hbm.at[idx.at[k]], bufs[k % 2], semg))
        if k >= 1:
            gath[k - 1].wait()
            orows = pl.ds(base + (k - 1) * CH, CH)
            pltpu.async_copy(bufs[(k - 1) % 2], out.at[orows], semw)
    gath[NCH - 1].wait()
    orows = pl.ds(base + (NCH - 1) * CH, CH)
    wlast = pltpu.async_copy(bufs[(NCH - 1) % 2], out.at[orows], semw)
    for k in range(NCH - 1):
        orows = pl.ds(base + k * CH, CH)
        pltpu.make_async_copy(bufs[k % 2], out.at[orows], semw).wait()
    wlast.wait()


N_ROWS = 100000
RC = 1000


def _concat_body(gu, mu, gi, mi, outu, outi):
    outu[:, :EMB] = gu[:].T
    outu[:, EMB:] = mu[:].T
    outi[:, :EMB] = gi[:].T
    outi[:, EMB:] = mi[:].T


_tc_concat = pl.pallas_call(
    _concat_body,
    grid=(N_ROWS // RC,),
    in_specs=[pl.BlockSpec((EMB, RC), lambda i: (0, i))] * 4,
    out_specs=[pl.BlockSpec((RC, 2 * EMB), lambda i: (i, 0))] * 2,
    out_shape=[jax.ShapeDtypeStruct((N_ROWS, 2 * EMB), jnp.float32)] * 2,
)


BB = 4096


def _tc_mlp_body(u, it, w1a, w1b, b1, w2, b2, w3, b3, wog, woh, bo, out):
    f32 = jnp.float32
    uu = u[:]
    ii = it[:]
    g = uu[:, :EMB] * ii[:, :EMB]
    acc = jnp.dot(g, wog[:], preferred_element_type=f32)
    h = jnp.dot(uu[:, EMB:], w1a[:], preferred_element_type=f32)
    h = h + jnp.dot(ii[:, EMB:], w1b[:], preferred_element_type=f32)
    h = jnp.maximum(h + b1[:], 0.0)
    h = jnp.maximum(jnp.dot(h, w2[:], preferred_element_type=f32) + b2[:], 0.0)
    h = jnp.maximum(jnp.dot(h, w3[:], preferred_element_type=f32) + b3[:], 0.0)
    out[:] = acc + jnp.dot(h, woh[:], preferred_element_type=f32) + bo[0, 0]


def _row_spec():
    return pl.BlockSpec((BB, 2 * EMB), lambda i: (i, 0))


def _full_spec(shape):
    return pl.BlockSpec(shape, lambda i: tuple(0 for _ in shape))


_tc_mlp = pl.pallas_call(
    _tc_mlp_body,
    grid=(BATCH // BB,),
    in_specs=[
        _row_spec(), _row_spec(),
        _full_spec((EMB, 128)), _full_spec((EMB, 128)), _full_spec((1, 128)),
        _full_spec((128, 64)), _full_spec((1, 64)),
        _full_spec((64, 32)), _full_spec((1, 32)),
        _full_spec((EMB, 1)), _full_spec((32, 1)), _full_spec((1, 1)),
    ],
    out_specs=pl.BlockSpec((BB, 1), lambda i: (i, 0)),
    out_shape=jax.ShapeDtypeStruct((BATCH, 1), jnp.float32),
)


@jax.jit
def kernel(user_ids, item_ids, gmf_user_emb, gmf_item_emb, mlp_user_emb,
           mlp_item_emb, W1, b1, W2, b2, W3, b3, Wo, bo):
    uid2d = user_ids.astype(jnp.int32).reshape(IDROWS, CH)
    iid2d = item_ids.astype(jnp.int32).reshape(IDROWS, CH)
    utab = jnp.concatenate([gmf_user_emb, mlp_user_emb], axis=1)
    itab = jnp.concatenate([gmf_item_emb, mlp_item_emb], axis=1)
    rows_u = _sc_gather(uid2d, utab)
    rows_i = _sc_gather(iid2d, itab)
    pred = _tc_mlp(rows_u, rows_i,
                   W1[:EMB], W1[EMB:], b1.reshape(1, -1),
                   W2, b2.reshape(1, -1), W3, b3.reshape(1, -1),
                   Wo[:EMB], Wo[EMB:], bo.reshape(1, 1))
    return pred.reshape(BATCH)
